# baseline (speedup 1.0000x reference)
import functools

import jax
import jax.numpy as jnp
from jax import lax
from jax.experimental import pallas as pl
from jax.experimental.pallas import tpu as pltpu
from jax.experimental.pallas import tpu_sc as plsc

N_SPEAKERS = 1000
BATCH = 16384

NUM_CORES = 2
NUM_SUBCORES = 16
LANES = 16
NUM_WORKERS = NUM_CORES * NUM_SUBCORES
COLS_PER_WORKER = BATCH // NUM_WORKERS
CHUNK_COLS = 128
COL_CHUNKS = COLS_PER_WORKER // CHUNK_COLS
BAND_ROWS = 200
ROW_BANDS = N_SPEAKERS // BAND_ROWS
COL_GROUPS = CHUNK_COLS // LANES
NBUF = 4
NUM_SLABS = COL_CHUNKS * ROW_BANDS

_mesh = plsc.VectorSubcoreMesh(
    core_axis_name="c",
    subcore_axis_name="s",
    num_cores=NUM_CORES,
    num_subcores=NUM_SUBCORES,
)


@functools.partial(
    pl.kernel,
    out_type=jax.ShapeDtypeStruct((N_SPEAKERS, BATCH), jnp.float32),
    mesh=_mesh,
    scratch_types=[
        pltpu.VMEM((COLS_PER_WORKER,), jnp.int32),
    ]
    + [pltpu.VMEM((BAND_ROWS, CHUNK_COLS), jnp.float32) for _ in range(NBUF)]
    + [pltpu.SemaphoreType.DMA for _ in range(NBUF)],
    compiler_params=pltpu.CompilerParams(
        needs_layout_passes=False,
        disable_bounds_checks=True,
        skip_device_barrier=True,
    ),
)
def _onehot_sc_t(ids_hbm, out_hbm, ids_v, *bufs_and_sems):
    bufs = bufs_and_sems[:NBUF]
    sems = bufs_and_sems[NBUF:]
    wid = lax.axis_index("s") * NUM_CORES + lax.axis_index("c")
    col_base = wid * COLS_PER_WORKER

    pltpu.sync_copy(ids_hbm.at[pl.ds(col_base, COLS_PER_WORKER)], ids_v)

    zeros = jnp.zeros((LANES,), jnp.float32)
    ones = jnp.ones((LANES,), jnp.float32)
    lane_iota = lax.iota(jnp.int32, LANES)

    def make_zero_body(buf):
        def zero_body(r, carry):
            for off in range(0, CHUNK_COLS, LANES):
                buf[r, pl.ds(off, LANES)] = zeros
            return carry

        return zero_body

    copies = [None] * NBUF
    prev = [None] * NBUF

    def scatter_slab(buf, cc, rb, value):
        r0 = rb * BAND_ROWS
        for g in range(COL_GROUPS):
            ids16 = ids_v[pl.ds(cc * CHUNK_COLS + g * LANES, LANES)]
            rows = ids16 - r0
            mask = (ids16 >= r0) & (ids16 < r0 + BAND_ROWS)
            cols = lane_iota + g * LANES
            plsc.store_scatter(buf, [rows, cols], value, mask=mask)

    for i in range(NUM_SLABS):
        cc, rb = divmod(i, ROW_BANDS)
        b = i % NBUF
        buf = bufs[b]
        if i < NBUF:
            lax.fori_loop(0, BAND_ROWS, make_zero_body(buf), 0)
        if copies[b] is not None:
            copies[b].wait()
            pcc, prb = prev[b]
            scatter_slab(buf, pcc, prb, zeros)
        scatter_slab(buf, cc, rb, ones)
        prev[b] = (cc, rb)
        dst = out_hbm.at[
            pl.ds(rb * BAND_ROWS, BAND_ROWS),
            pl.ds(col_base + cc * CHUNK_COLS, CHUNK_COLS),
        ]
        copies[b] = pltpu.async_copy(buf, dst, sems[b])

    for b in range(NBUF):
        if copies[b] is not None:
            copies[b].wait()


def kernel(style_id):
    return _onehot_sc_t(style_id).T


# speedup vs baseline: 1.0168x; 1.0168x over previous
import functools

import jax
import jax.numpy as jnp
from jax import lax
from jax.experimental import pallas as pl
from jax.experimental.pallas import tpu as pltpu
from jax.experimental.pallas import tpu_sc as plsc

N_SPEAKERS = 1000
BATCH = 16384

NUM_CORES = 2
NUM_SUBCORES = 16
LANES = 16
NUM_WORKERS = NUM_CORES * NUM_SUBCORES
COLS_PER_WORKER = BATCH // NUM_WORKERS
CHUNK_COLS = 128
COL_CHUNKS = COLS_PER_WORKER // CHUNK_COLS
BANDS = (504, 496)
BAND_STARTS = (0, 504)
BUF_ROWS = max(BANDS)
COL_GROUPS = CHUNK_COLS // LANES

_mesh = plsc.VectorSubcoreMesh(
    core_axis_name="c",
    subcore_axis_name="s",
    num_cores=NUM_CORES,
    num_subcores=NUM_SUBCORES,
)


@functools.partial(
    pl.kernel,
    out_type=jax.ShapeDtypeStruct((N_SPEAKERS, BATCH), jnp.float32),
    mesh=_mesh,
    scratch_types=[
        pltpu.VMEM((COLS_PER_WORKER,), jnp.int32),
        pltpu.VMEM((BUF_ROWS, CHUNK_COLS), jnp.float32),
        pltpu.VMEM((BUF_ROWS, CHUNK_COLS), jnp.float32),
        pltpu.SemaphoreType.DMA,
        pltpu.SemaphoreType.DMA,
    ],
    compiler_params=pltpu.CompilerParams(
        needs_layout_passes=False,
        disable_bounds_checks=True,
        skip_device_barrier=True,
    ),
)
def _onehot_sc_t(ids_hbm, out_hbm, ids_v, buf0, buf1, sem0, sem1):
    wid = lax.axis_index("s") * NUM_CORES + lax.axis_index("c")
    col_base = wid * COLS_PER_WORKER

    pltpu.sync_copy(ids_hbm.at[pl.ds(col_base, COLS_PER_WORKER)], ids_v)

    zeros = jnp.zeros((LANES,), jnp.float32)
    ones = jnp.ones((LANES,), jnp.float32)
    lane_iota = lax.iota(jnp.int32, LANES)

    def make_zero_body(buf):
        def zero_body(r, carry):
            for off in range(0, CHUNK_COLS, LANES):
                buf[r, pl.ds(off, LANES)] = zeros
            return carry

        return zero_body

    bufs = (buf0, buf1)
    sems = (sem0, sem1)
    copies = [None, None]
    prev = [None, None]

    def scatter_slab(buf, cc, rb, value):
        r0 = BAND_STARTS[rb]
        h = BANDS[rb]
        for g in range(COL_GROUPS):
            ids16 = ids_v[pl.ds(cc * CHUNK_COLS + g * LANES, LANES)]
            rows = ids16 - r0
            mask = (ids16 >= r0) & (ids16 < r0 + h)
            cols = lane_iota + g * LANES
            plsc.store_scatter(buf, [rows, cols], value, mask=mask)

    for i in range(COL_CHUNKS * len(BANDS)):
        cc, rb = divmod(i, len(BANDS))
        b = i % 2
        buf = bufs[b]
        if i < 2:
            lax.fori_loop(0, BUF_ROWS, make_zero_body(buf), 0)
        if copies[b] is not None:
            copies[b].wait()
            pcc, prb = prev[b]
            scatter_slab(buf, pcc, prb, zeros)
        scatter_slab(buf, cc, rb, ones)
        prev[b] = (cc, rb)
        h = BANDS[rb]
        dst = out_hbm.at[
            pl.ds(BAND_STARTS[rb], h),
            pl.ds(col_base + cc * CHUNK_COLS, CHUNK_COLS),
        ]
        copies[b] = pltpu.async_copy(buf.at[pl.ds(0, h)], dst, sems[b])

    for b in range(2):
        if copies[b] is not None:
            copies[b].wait()


def kernel(style_id):
    return _onehot_sc_t(style_id).T


# speedup vs baseline: 1.0195x; 1.0027x over previous
import functools

import jax
import jax.numpy as jnp
from jax import lax
from jax.experimental import pallas as pl
from jax.experimental.pallas import tpu as pltpu
from jax.experimental.pallas import tpu_sc as plsc

N_SPEAKERS = 1000
BATCH = 16384

NUM_CORES = 2
NUM_SUBCORES = 16
LANES = 16
NUM_WORKERS = NUM_CORES * NUM_SUBCORES
COLS_PER_WORKER = BATCH // NUM_WORKERS
CHUNK_COLS = 128
COL_CHUNKS = COLS_PER_WORKER // CHUNK_COLS
BANDS = (104, 448, 448)
BAND_STARTS = (0, 104, 552)
COL_GROUPS = CHUNK_COLS // LANES

_mesh = plsc.VectorSubcoreMesh(
    core_axis_name="c",
    subcore_axis_name="s",
    num_cores=NUM_CORES,
    num_subcores=NUM_SUBCORES,
)


@functools.partial(
    pl.kernel,
    out_type=jax.ShapeDtypeStruct((N_SPEAKERS, BATCH), jnp.float32),
    mesh=_mesh,
    scratch_types=[
        pltpu.VMEM((COLS_PER_WORKER,), jnp.int32),
        pltpu.VMEM((BANDS[0], CHUNK_COLS), jnp.float32),
        pltpu.VMEM((BANDS[1], CHUNK_COLS), jnp.float32),
        pltpu.VMEM((BANDS[2], CHUNK_COLS), jnp.float32),
        pltpu.SemaphoreType.DMA,
        pltpu.SemaphoreType.DMA,
        pltpu.SemaphoreType.DMA,
        pltpu.SemaphoreType.DMA,
    ],
    compiler_params=pltpu.CompilerParams(
        needs_layout_passes=False,
        disable_bounds_checks=True,
        skip_device_barrier=True,
    ),
)
def _onehot_sc_t(
    ids_hbm, out_hbm, ids_v, buf0, buf1, buf2, sem_ids, sem0, sem1, sem2
):
    wid = lax.axis_index("s") * NUM_CORES + lax.axis_index("c")
    col_base = wid * COLS_PER_WORKER

    ids_copy = pltpu.async_copy(
        ids_hbm.at[pl.ds(col_base, COLS_PER_WORKER)], ids_v, sem_ids
    )

    zeros = jnp.zeros((LANES,), jnp.float32)
    ones = jnp.ones((LANES,), jnp.float32)
    lane_iota = lax.iota(jnp.int32, LANES)

    def make_zero_body(buf):
        def zero_body(r, carry):
            for off in range(0, CHUNK_COLS, LANES):
                buf[r, pl.ds(off, LANES)] = zeros
            return carry

        return zero_body

    bufs = (buf0, buf1, buf2)
    sems = (sem0, sem1, sem2)
    copies = [None, None, None]

    def scatter_slab(buf, cc, rb, value):
        r0 = BAND_STARTS[rb]
        h = BANDS[rb]
        for g in range(COL_GROUPS):
            ids16 = ids_v[pl.ds(cc * CHUNK_COLS + g * LANES, LANES)]
            rows = ids16 - r0
            mask = (ids16 >= r0) & (ids16 < r0 + h)
            cols = lane_iota + g * LANES
            plsc.store_scatter(buf, [rows, cols], value, mask=mask)

    for i in range(COL_CHUNKS * len(BANDS)):
        cc, rb = divmod(i, len(BANDS))
        buf = bufs[rb]
        if cc == 0:
            lax.fori_loop(0, BANDS[rb], make_zero_body(buf), 0)
            if rb == 0:
                ids_copy.wait()
        if copies[rb] is not None:
            copies[rb].wait()
            scatter_slab(buf, cc - 1, rb, zeros)
        scatter_slab(buf, cc, rb, ones)
        dst = out_hbm.at[
            pl.ds(BAND_STARTS[rb], BANDS[rb]),
            pl.ds(col_base + cc * CHUNK_COLS, CHUNK_COLS),
        ]
        copies[rb] = pltpu.async_copy(buf, dst, sems[rb])

    for rb in range(len(BANDS)):
        if copies[rb] is not None:
            copies[rb].wait()


def kernel(style_id):
    return _onehot_sc_t(style_id).T


# speedup vs baseline: 1.0290x; 1.0093x over previous
import functools

import jax
import jax.numpy as jnp
from jax import lax
from jax.experimental import pallas as pl
from jax.experimental.pallas import tpu as pltpu
from jax.experimental.pallas import tpu_sc as plsc

N_SPEAKERS = 1000
BATCH = 16384

NUM_CORES = 2
NUM_SUBCORES = 16
LANES = 16
NUM_WORKERS = NUM_CORES * NUM_SUBCORES
COLS_PER_WORKER = BATCH // NUM_WORKERS
BAND_ROWS = 48
NUM_BANDS = 21
LAST_BAND_ROWS = N_SPEAKERS - (NUM_BANDS - 1) * BAND_ROWS
COL_GROUPS = COLS_PER_WORKER // LANES
NBUF = 3

_mesh = plsc.VectorSubcoreMesh(
    core_axis_name="c",
    subcore_axis_name="s",
    num_cores=NUM_CORES,
    num_subcores=NUM_SUBCORES,
)


@functools.partial(
    pl.kernel,
    out_type=jax.ShapeDtypeStruct((N_SPEAKERS, BATCH), jnp.float32),
    mesh=_mesh,
    scratch_types=[
        pltpu.VMEM((COLS_PER_WORKER,), jnp.int32),
    ]
    + [
        pltpu.VMEM((BAND_ROWS, COLS_PER_WORKER), jnp.float32)
        for _ in range(NBUF)
    ]
    + [pltpu.SemaphoreType.DMA for _ in range(NBUF + 1)],
    compiler_params=pltpu.CompilerParams(
        needs_layout_passes=False,
        disable_bounds_checks=True,
        skip_device_barrier=True,
    ),
)
def _onehot_sc_t(ids_hbm, out_hbm, ids_v, *bufs_and_sems):
    bufs = bufs_and_sems[:NBUF]
    sems = bufs_and_sems[NBUF : 2 * NBUF]
    sem_ids = bufs_and_sems[2 * NBUF]
    wid = lax.axis_index("s") * NUM_CORES + lax.axis_index("c")
    col_base = wid * COLS_PER_WORKER

    ids_copy = pltpu.async_copy(
        ids_hbm.at[pl.ds(col_base, COLS_PER_WORKER)], ids_v, sem_ids
    )

    zeros = jnp.zeros((LANES,), jnp.float32)
    ones = jnp.ones((LANES,), jnp.float32)
    lane_iota = lax.iota(jnp.int32, LANES)

    def make_zero_body(buf):
        def zero_body(r, carry):
            for off in range(0, COLS_PER_WORKER, LANES):
                buf[r, pl.ds(off, LANES)] = zeros
            return carry

        return zero_body

    copies = [None] * NBUF
    prev = [None] * NBUF

    def scatter_slab(buf, band, value):
        r0 = band * BAND_ROWS
        h = LAST_BAND_ROWS if band == NUM_BANDS - 1 else BAND_ROWS

        def group_body(g, carry):
            ids16 = ids_v[pl.ds(g * LANES, LANES)]
            rows = ids16 - r0
            mask = (ids16 >= r0) & (ids16 < r0 + h)
            cols = lane_iota + g * LANES
            plsc.store_scatter(buf, [rows, cols], value, mask=mask)
            return carry

        lax.fori_loop(0, COL_GROUPS, group_body, 0)

    for band in range(NUM_BANDS):
        b = band % NBUF
        buf = bufs[b]
        if band < NBUF:
            lax.fori_loop(0, BAND_ROWS, make_zero_body(buf), 0)
            if band == 0:
                ids_copy.wait()
        if copies[b] is not None:
            copies[b].wait()
            scatter_slab(buf, prev[b], zeros)
        scatter_slab(buf, band, ones)
        prev[b] = band
        h = LAST_BAND_ROWS if band == NUM_BANDS - 1 else BAND_ROWS
        dst = out_hbm.at[
            pl.ds(band * BAND_ROWS, h),
            pl.ds(col_base, COLS_PER_WORKER),
        ]
        copies[b] = pltpu.async_copy(buf.at[pl.ds(0, h)], dst, sems[b])

    for b in range(NBUF):
        if copies[b] is not None:
            copies[b].wait()


def kernel(style_id):
    return _onehot_sc_t(style_id).T


# speedup vs baseline: 1.0403x; 1.0110x over previous
import functools

import jax
import jax.numpy as jnp
from jax import lax
from jax.experimental import pallas as pl
from jax.experimental.pallas import tpu as pltpu
from jax.experimental.pallas import tpu_sc as plsc

N_SPEAKERS = 1000
BATCH = 16384

NUM_CORES = 2
NUM_SUBCORES = 16
LANES = 16
NUM_WORKERS = NUM_CORES * NUM_SUBCORES
COLS_PER_WORKER = BATCH // NUM_WORKERS
BAND_ROWS = 80
NUM_BANDS = 13
LAST_BAND_ROWS = N_SPEAKERS - (NUM_BANDS - 1) * BAND_ROWS
COL_GROUPS = COLS_PER_WORKER // LANES
NBUF = 3

_mesh = plsc.VectorSubcoreMesh(
    core_axis_name="c",
    subcore_axis_name="s",
    num_cores=NUM_CORES,
    num_subcores=NUM_SUBCORES,
)


@functools.partial(
    pl.kernel,
    out_type=jax.ShapeDtypeStruct((N_SPEAKERS, BATCH), jnp.float32),
    mesh=_mesh,
    scratch_types=[
        pltpu.VMEM((COLS_PER_WORKER,), jnp.int32),
    ]
    + [
        pltpu.VMEM((BAND_ROWS, COLS_PER_WORKER), jnp.float32)
        for _ in range(NBUF)
    ]
    + [pltpu.SemaphoreType.DMA for _ in range(NBUF + 1)],
    compiler_params=pltpu.CompilerParams(
        needs_layout_passes=False,
        disable_bounds_checks=True,
        skip_device_barrier=True,
    ),
)
def _onehot_sc_t(ids_hbm, out_hbm, ids_v, *bufs_and_sems):
    bufs = bufs_and_sems[:NBUF]
    sems = bufs_and_sems[NBUF : 2 * NBUF]
    sem_ids = bufs_and_sems[2 * NBUF]
    wid = lax.axis_index("s") * NUM_CORES + lax.axis_index("c")
    col_base = wid * COLS_PER_WORKER

    ids_copy = pltpu.async_copy(
        ids_hbm.at[pl.ds(col_base, COLS_PER_WORKER)], ids_v, sem_ids
    )

    zeros = jnp.zeros((LANES,), jnp.float32)
    ones = jnp.ones((LANES,), jnp.float32)
    lane_iota = lax.iota(jnp.int32, LANES)

    def make_zero_body(buf):
        def zero_body(r, carry):
            for off in range(0, COLS_PER_WORKER, LANES):
                buf[r, pl.ds(off, LANES)] = zeros
            return carry

        return zero_body

    copies = [None] * NBUF
    prev = [None] * NBUF

    def scatter_slab(buf, band, value):
        r0 = band * BAND_ROWS
        h = LAST_BAND_ROWS if band == NUM_BANDS - 1 else BAND_ROWS

        def group_body(g, carry):
            ids16 = ids_v[pl.ds(g * LANES, LANES)]
            rows = ids16 - r0
            mask = (ids16 >= r0) & (ids16 < r0 + h)
            cols = lane_iota + g * LANES
            plsc.store_scatter(buf, [rows, cols], value, mask=mask)
            return carry

        lax.fori_loop(0, COL_GROUPS, group_body, 0)

    for band in range(NUM_BANDS):
        b = band % NBUF
        buf = bufs[b]
        if band < NBUF:
            lax.fori_loop(0, BAND_ROWS, make_zero_body(buf), 0)
            if band == 0:
                ids_copy.wait()
        if copies[b] is not None:
            copies[b].wait()
            scatter_slab(buf, prev[b], zeros)
        scatter_slab(buf, band, ones)
        prev[b] = band
        h = LAST_BAND_ROWS if band == NUM_BANDS - 1 else BAND_ROWS
        dst = out_hbm.at[
            pl.ds(band * BAND_ROWS, h),
            pl.ds(col_base, COLS_PER_WORKER),
        ]
        copies[b] = pltpu.async_copy(buf.at[pl.ds(0, h)], dst, sems[b])

    for b in range(NBUF):
        if copies[b] is not None:
            copies[b].wait()


def kernel(style_id):
    return _onehot_sc_t(style_id).T
